# Initial kernel scaffold; baseline (speedup 1.0000x reference)
import functools

import jax
import jax.numpy as jnp
from jax import lax
from jax.experimental import pallas as pl
from jax.experimental.pallas import tpu as pltpu
from jax.experimental.pallas import tpu_sc as plsc

_VOCAB = 100000
_SEQ = 200
_D = 64
_C = 10000
_BATCH = 4096

_NC = 2
_NS = 16
_NW = _NC * _NS

_ROWS = _BATCH * _SEQ
_RPW = _ROWS // _NW
_CHUNK = 400
_NCHUNK = _RPW // _CHUNK
_G = 100
_NG = _CHUNK // _G
_LANES = 16
_VPR = _D // _LANES


def _positional(seq_len, d_model, c):
    pos = jnp.arange(1, seq_len + 1, dtype=jnp.float32)[:, None]
    j = jnp.arange(d_model)[None, :]
    k = (j + 1) // 2
    angle = pos / jnp.power(jnp.float32(c), k.astype(jnp.float32) / d_model)
    return jnp.where((j % 2) == 0, jnp.sin(angle), jnp.cos(angle)).astype(
        jnp.float32
    )


@functools.partial(
    pl.kernel,
    out_type=jax.ShapeDtypeStruct((_ROWS, _D), jnp.float32),
    mesh=plsc.VectorSubcoreMesh(core_axis_name="c", subcore_axis_name="s"),
    scratch_types=[
        pltpu.VMEM((_SEQ, _D), jnp.float32),
        pltpu.VMEM((_NG, _G), jnp.int32),
        pltpu.VMEM((_CHUNK, _D), jnp.float32),
        pltpu.SemaphoreType.DMA,
    ],
)
def _emb_lookup(idx_hbm, table_hbm, pos_hbm, out_hbm, pos_v, idx_v, rows_v,
                sem):
    wid = lax.axis_index("s") * _NC + lax.axis_index("c")
    base = wid * _RPW
    pltpu.sync_copy(pos_hbm, pos_v)

    def chunk_body(ci, carry):
        rbase = base + ci * _CHUNK
        pltpu.sync_copy(idx_hbm.at[pl.ds(rbase // _G, _NG)], idx_v)
        copies = []
        for g in range(_NG):
            copies.append(
                pltpu.async_copy(
                    table_hbm.at[idx_v.at[g]],
                    rows_v.at[pl.ds(g * _G, _G)],
                    sem,
                )
            )
        for c in copies:
            c.wait()

        def add_seq(s):
            def row_body(r, acc):
                for cvr in range(_VPR):
                    sl = pl.ds(cvr * _LANES, _LANES)
                    rows_v[s * _SEQ + r, sl] = (
                        rows_v[s * _SEQ + r, sl] + pos_v[r, sl]
                    )
                return acc

            lax.fori_loop(0, _SEQ, row_body, 0)

        for s in range(_CHUNK // _SEQ):
            add_seq(s)

        pltpu.sync_copy(rows_v, out_hbm.at[pl.ds(rbase, _CHUNK)])
        return carry

    lax.fori_loop(0, _NCHUNK, chunk_body, 0)


def kernel(x, table):
    idx = x.reshape(_ROWS // _G, _G).astype(jnp.int32)
    pos = _positional(_SEQ, _D, _C)
    out = _emb_lookup(idx, table, pos)
    return out.reshape(_BATCH, _SEQ, _D)


# speedup vs baseline: 3.5040x; 3.5040x over previous
import functools

import jax
import jax.numpy as jnp
from jax import lax
from jax.experimental import pallas as pl
from jax.experimental.pallas import tpu as pltpu
from jax.experimental.pallas import tpu_sc as plsc

_VOCAB = 100000
_SEQ = 200
_D = 64
_C = 10000
_BATCH = 4096

_NC = 2
_NS = 16
_NW = _NC * _NS

_ROWS = _BATCH * _SEQ
_RPW = _ROWS // _NW
_CHUNK = 800
_NCHUNK = _RPW // _CHUNK
_G = 100
_NG = _CHUNK // _G
_LANES = 16
_VPR = _D // _LANES


def _positional(seq_len, d_model, c):
    pos = jnp.arange(1, seq_len + 1, dtype=jnp.float32)[:, None]
    j = jnp.arange(d_model)[None, :]
    k = (j + 1) // 2
    angle = pos / jnp.power(jnp.float32(c), k.astype(jnp.float32) / d_model)
    return jnp.where((j % 2) == 0, jnp.sin(angle), jnp.cos(angle)).astype(
        jnp.float32
    )


@functools.partial(
    pl.kernel,
    out_type=jax.ShapeDtypeStruct((_ROWS, _D), jnp.float32),
    mesh=plsc.VectorSubcoreMesh(core_axis_name="c", subcore_axis_name="s"),
    scratch_types=[
        pltpu.VMEM((_SEQ, _D), jnp.float32),
        pltpu.VMEM((_NG, _G), jnp.int32),
        pltpu.VMEM((_CHUNK, _D), jnp.float32),
        pltpu.SemaphoreType.DMA,
    ],
    compiler_params=pltpu.CompilerParams(use_tc_tiling_on_sc=False),
)
def _emb_lookup(idx_hbm, table_hbm, pos_hbm, out_hbm, pos_v, idx_v, rows_v,
                sem):
    wid = lax.axis_index("s") * _NC + lax.axis_index("c")
    base = wid * _RPW
    pltpu.sync_copy(pos_hbm, pos_v)

    def chunk_body(ci, carry):
        rbase = pl.multiple_of(base + ci * _CHUNK, _CHUNK)
        pltpu.sync_copy(
            idx_hbm.at[pl.ds(pl.multiple_of(rbase // _G, 8), _NG)], idx_v
        )
        copies = []
        for g in range(_NG):
            copies.append(
                pltpu.async_copy(
                    table_hbm.at[idx_v.at[g]],
                    rows_v.at[pl.ds(g * _G, _G)],
                    sem,
                )
            )
        for c in copies:
            c.wait()

        def add_seq(s):
            def row_body(r, acc):
                for cvr in range(_VPR):
                    sl = pl.ds(cvr * _LANES, _LANES)
                    rows_v[s * _SEQ + r, sl] = (
                        rows_v[s * _SEQ + r, sl] + pos_v[r, sl]
                    )
                return acc

            lax.fori_loop(0, _SEQ, row_body, 0)

        for s in range(_CHUNK // _SEQ):
            add_seq(s)

        pltpu.sync_copy(rows_v, out_hbm.at[pl.ds(rbase, _CHUNK)])
        return carry

    lax.fori_loop(0, _NCHUNK, chunk_body, 0)


def kernel(x, table):
    idx = x.reshape(_ROWS // _G, _G).astype(jnp.int32)
    pos = _positional(_SEQ, _D, _C)
    out = _emb_lookup(idx, table, pos)
    return out.reshape(_BATCH, _SEQ, _D)


# speedup vs baseline: 4.1677x; 1.1894x over previous
import functools

import jax
import jax.numpy as jnp
from jax import lax
from jax.experimental import pallas as pl
from jax.experimental.pallas import tpu as pltpu
from jax.experimental.pallas import tpu_sc as plsc

_VOCAB = 100000
_SEQ = 200
_D = 64
_C = 10000
_BATCH = 4096

_NC = 2
_NS = 16
_NW = _NC * _NS

_ROWS = _BATCH * _SEQ
_RPW = _ROWS // _NW
_CHUNK = 400
_NCHUNK = _RPW // _CHUNK
_PAIRS = _NCHUNK // 2
_G = 80
_NG = _CHUNK // _G
_LANES = 16
_VPR = _D // _LANES


def _positional(seq_len, d_model, c):
    pos = jnp.arange(1, seq_len + 1, dtype=jnp.float32)[:, None]
    j = jnp.arange(d_model)[None, :]
    k = (j + 1) // 2
    angle = pos / jnp.power(jnp.float32(c), k.astype(jnp.float32) / d_model)
    return jnp.where((j % 2) == 0, jnp.sin(angle), jnp.cos(angle)).astype(
        jnp.float32
    )


@functools.partial(
    pl.kernel,
    out_type=jax.ShapeDtypeStruct((_ROWS, _D), jnp.float32),
    mesh=plsc.VectorSubcoreMesh(core_axis_name="c", subcore_axis_name="s"),
    scratch_types=[
        pltpu.VMEM((_SEQ, _D), jnp.float32),
        pltpu.VMEM((_RPW,), jnp.int32),
        pltpu.VMEM((_CHUNK, _D), jnp.float32),
        pltpu.VMEM((_CHUNK, _D), jnp.float32),
        pltpu.SemaphoreType.DMA,
        pltpu.SemaphoreType.DMA,
        pltpu.SemaphoreType.DMA,
        pltpu.SemaphoreType.DMA,
    ],
    compiler_params=pltpu.CompilerParams(use_tc_tiling_on_sc=False),
)
def _emb_lookup(idx_hbm, table_hbm, pos_hbm, out_hbm, pos_v, idx_v, rows0,
                rows1, gsem0, gsem1, osem0, osem1):
    wid = lax.axis_index("s") * _NC + lax.axis_index("c")
    base = pl.multiple_of(wid * _RPW, _RPW)
    pltpu.sync_copy(pos_hbm, pos_v)
    pltpu.sync_copy(idx_hbm.at[pl.ds(base, _RPW)], idx_v)

    def fire_gather(ci, rows, sem):
        off = pl.multiple_of(ci * _CHUNK, _G)
        for g in range(_NG):
            pltpu.async_copy(
                table_hbm.at[idx_v.at[pl.ds(pl.multiple_of(off + g * _G, 8),
                                            _G)]],
                rows.at[pl.ds(g * _G, _G)],
                sem,
            )

    def wait_gather(rows, sem):
        for g in range(_NG):
            pltpu.make_async_copy(
                table_hbm.at[idx_v.at[pl.ds(g * _G, _G)]],
                rows.at[pl.ds(g * _G, _G)],
                sem,
            ).wait()

    def fire_out(ci, rows, sem):
        roff = pl.multiple_of(base + ci * _CHUNK, _CHUNK)
        pltpu.async_copy(rows, out_hbm.at[pl.ds(roff, _CHUNK)], sem)

    def wait_out(rows, sem):
        pltpu.make_async_copy(rows, out_hbm.at[pl.ds(0, _CHUNK)], sem).wait()

    def add_pos(rows):
        @plsc.parallel_loop(0, _SEQ, unroll=2)
        def _(r):
            for c in range(_VPR):
                sl = pl.ds(c * _LANES, _LANES)
                p = pos_v[r, sl]
                rows[r, sl] = rows[r, sl] + p
                rows[_SEQ + r, sl] = rows[_SEQ + r, sl] + p

    fire_gather(0, rows0, gsem0)

    def pair_body(k, acc):
        e = k * 2

        @pl.when(k > 0)
        def _():
            wait_out(rows1, osem1)

        fire_gather(e + 1, rows1, gsem1)

        wait_gather(rows0, gsem0)
        add_pos(rows0)
        fire_out(e, rows0, osem0)

        @pl.when(k < _PAIRS - 1)
        def _():
            wait_out(rows0, osem0)
            fire_gather(e + 2, rows0, gsem0)

        wait_gather(rows1, gsem1)
        add_pos(rows1)
        fire_out(e + 1, rows1, osem1)
        return acc

    lax.fori_loop(0, _PAIRS, pair_body, 0)
    wait_out(rows0, osem0)
    wait_out(rows1, osem1)


def kernel(x, table):
    idx = x.reshape(_ROWS).astype(jnp.int32)
    pos = _positional(_SEQ, _D, _C)
    out = _emb_lookup(idx, table, pos)
    return out.reshape(_BATCH, _SEQ, _D)


# speedup vs baseline: 4.2281x; 1.0145x over previous
import functools

import jax
import jax.numpy as jnp
from jax import lax
from jax.experimental import pallas as pl
from jax.experimental.pallas import tpu as pltpu
from jax.experimental.pallas import tpu_sc as plsc

_VOCAB = 100000
_SEQ = 200
_D = 64
_C = 10000
_BATCH = 4096

_NC = 2
_NS = 16
_NW = _NC * _NS

_ROWS = _BATCH * _SEQ
_RPW = _ROWS // _NW
_CHUNK = 400
_NCHUNK = _RPW // _CHUNK
_PAIRS = _NCHUNK // 2
_G = 80
_NG = _CHUNK // _G
_LANES = 16
_VPR = _D // _LANES
_OC = _CHUNK * _D // 128
_HSEQ = _SEQ // 2


def _positional(seq_len, d_model, c):
    pos = jnp.arange(1, seq_len + 1, dtype=jnp.float32)[:, None]
    j = jnp.arange(d_model)[None, :]
    k = (j + 1) // 2
    angle = pos / jnp.power(jnp.float32(c), k.astype(jnp.float32) / d_model)
    return jnp.where((j % 2) == 0, jnp.sin(angle), jnp.cos(angle)).astype(
        jnp.float32
    )


@functools.partial(
    pl.kernel,
    out_type=jax.ShapeDtypeStruct((_ROWS * _D // 128, 128), jnp.float32),
    mesh=plsc.VectorSubcoreMesh(core_axis_name="c", subcore_axis_name="s"),
    scratch_types=[
        pltpu.VMEM((_SEQ, _D), jnp.float32),
        pltpu.VMEM((_CHUNK,), jnp.int32),
        pltpu.VMEM((_CHUNK,), jnp.int32),
        pltpu.VMEM((_CHUNK, _D), jnp.float32),
        pltpu.VMEM((_CHUNK, _D), jnp.float32),
        pltpu.VMEM((_OC, 128), jnp.float32),
        pltpu.VMEM((_OC, 128), jnp.float32),
        pltpu.SemaphoreType.DMA,
        pltpu.SemaphoreType.DMA,
        pltpu.SemaphoreType.DMA,
        pltpu.SemaphoreType.DMA,
        pltpu.SemaphoreType.DMA,
        pltpu.SemaphoreType.DMA,
    ],
    compiler_params=pltpu.CompilerParams(use_tc_tiling_on_sc=False),
)
def _emb_lookup(idx_hbm, table_hbm, pos_hbm, out_hbm, pos_v, idx0, idx1,
                gbuf0, gbuf1, obuf0, obuf1, isem0, isem1, gsem0, gsem1,
                osem0, osem1):
    wid = lax.axis_index("s") * _NC + lax.axis_index("c")
    base = pl.multiple_of(wid * _RPW, _RPW)
    obase = pl.multiple_of(wid * (_RPW * _D // 128), _RPW * _D // 128)
    pltpu.sync_copy(pos_hbm, pos_v)

    def fire_idx(ci, idxv, sem):
        off = pl.multiple_of(base + ci * _CHUNK, _CHUNK)
        pltpu.async_copy(idx_hbm.at[pl.ds(off, _CHUNK)], idxv, sem)

    def wait_idx(idxv, sem):
        pltpu.make_async_copy(idx_hbm.at[pl.ds(0, _CHUNK)], idxv, sem).wait()

    def fire_gather(idxv, gbuf, sem):
        for g in range(_NG):
            pltpu.async_copy(
                table_hbm.at[idxv.at[pl.ds(g * _G, _G)]],
                gbuf.at[pl.ds(g * _G, _G)],
                sem,
            )

    def wait_gather(gbuf, sem):
        for g in range(_NG):
            pltpu.make_async_copy(
                table_hbm.at[idx0.at[pl.ds(g * _G, _G)]],
                gbuf.at[pl.ds(g * _G, _G)],
                sem,
            ).wait()

    def fire_out(ci, obuf, sem):
        roff = pl.multiple_of(obase + ci * _OC, _OC)
        pltpu.async_copy(obuf, out_hbm.at[pl.ds(roff, _OC)], sem)

    def wait_out(obuf, sem):
        pltpu.make_async_copy(obuf, out_hbm.at[pl.ds(0, _OC)], sem).wait()

    def add_pos(gbuf, obuf):
        @plsc.parallel_loop(0, _HSEQ, unroll=2)
        def _(k):
            r2 = pl.multiple_of(k * 2, 2)
            for c in range(_VPR):
                sl = pl.ds(c * _LANES, _LANES)
                sh = pl.ds(_D + c * _LANES, _LANES)
                pa = pos_v[r2, sl]
                pb = pos_v[r2 + 1, sl]
                obuf[k, sl] = gbuf[r2, sl] + pa
                obuf[k, sh] = gbuf[r2 + 1, sl] + pb
                obuf[_HSEQ + k, sl] = gbuf[_SEQ + r2, sl] + pa
                obuf[_HSEQ + k, sh] = gbuf[_SEQ + r2 + 1, sl] + pb

    fire_idx(0, idx0, isem0)
    wait_idx(idx0, isem0)
    fire_gather(idx0, gbuf0, gsem0)
    fire_idx(1, idx1, isem1)

    def pair_body(k, acc):
        e = k * 2

        wait_idx(idx1, isem1)
        fire_gather(idx1, gbuf1, gsem1)
        wait_gather(gbuf0, gsem0)

        @pl.when(k < _PAIRS - 1)
        def _():
            fire_idx(e + 2, idx0, isem0)

        @pl.when(k > 0)
        def _():
            wait_out(obuf0, osem0)

        add_pos(gbuf0, obuf0)
        fire_out(e, obuf0, osem0)

        @pl.when(k < _PAIRS - 1)
        def _():
            wait_idx(idx0, isem0)
            fire_gather(idx0, gbuf0, gsem0)

        wait_gather(gbuf1, gsem1)

        @pl.when(k < _PAIRS - 1)
        def _():
            fire_idx(e + 3, idx1, isem1)

        @pl.when(k > 0)
        def _():
            wait_out(obuf1, osem1)

        add_pos(gbuf1, obuf1)
        fire_out(e + 1, obuf1, osem1)
        return acc

    lax.fori_loop(0, _PAIRS, pair_body, 0)
    wait_out(obuf0, osem0)
    wait_out(obuf1, osem1)


def kernel(x, table):
    idx = x.reshape(_ROWS).astype(jnp.int32)
    pos = _positional(_SEQ, _D, _C)
    out = _emb_lookup(idx, table, pos)
    return out.reshape(_BATCH, _SEQ, _D)
